# baseline (speedup 1.0000x reference)
import functools

import jax
import jax.numpy as jnp
from jax import lax
from jax.experimental import pallas as pl
from jax.experimental.pallas import tpu as pltpu
from jax.experimental.pallas import tpu_sc as plsc

B = 16384
D = 32

_NC = 2
_NS = 16
_NW = _NC * _NS
_BPW = B // _NW
_CH = 128
_NCH = _BPW // _CH


def _sc_gather3(uid2, iid2, pid2, ut, it, pt):
    dt = ut.dtype
    mesh = plsc.VectorSubcoreMesh(core_axis_name="c", subcore_axis_name="s")
    out_t = jax.ShapeDtypeStruct((B, D), dt)

    @functools.partial(
        pl.kernel,
        out_type=[out_t, out_t, out_t],
        mesh=mesh,
        compiler_params=pltpu.CompilerParams(use_tc_tiling_on_sc=False),
        scratch_types=[
            pltpu.VMEM((_BPW,), jnp.int32),
            pltpu.VMEM((_BPW,), jnp.int32),
            pltpu.VMEM((_BPW,), jnp.int32),
            pltpu.VMEM((_BPW, D), dt),
            pltpu.VMEM((_BPW, D), dt),
            pltpu.VMEM((_BPW, D), dt),
            pltpu.SemaphoreType.DMA,
        ],
    )
    def k(uid_h, iid_h, pid_h, ut_h, it_h, pt_h, ou_h, oi_h, op_h,
          idx_u, idx_i, idx_p, rw_u, rw_i, rw_p, sem):
        wid = lax.axis_index("s") * _NC + lax.axis_index("c")
        base = wid * _BPW
        pltpu.sync_copy(uid_h.at[wid], idx_u)
        pltpu.sync_copy(iid_h.at[wid], idx_i)
        pltpu.sync_copy(pid_h.at[wid], idx_p)
        copies = []
        for j in range(_NCH):
            sl = pl.ds(j * _CH, _CH)
            copies.append(
                pltpu.async_copy(ut_h.at[idx_u.at[sl]], rw_u.at[sl], sem))
            copies.append(
                pltpu.async_copy(it_h.at[idx_i.at[sl]], rw_i.at[sl], sem))
            copies.append(
                pltpu.async_copy(pt_h.at[idx_p.at[sl]], rw_p.at[sl], sem))
        for c in copies:
            c.wait()
        pltpu.sync_copy(rw_u, ou_h.at[pl.ds(base, _BPW)])
        pltpu.sync_copy(rw_i, oi_h.at[pl.ds(base, _BPW)])
        pltpu.sync_copy(rw_p, op_h.at[pl.ds(base, _BPW)])

    return k(uid2, iid2, pid2, ut, it, pt)


_BT = 2048
_NB = B // _BT


def _tc_body(ue_r, ie_r, pe_r,
             la_r, eb_r, fm_r, de_r, av_r, pg_r,
             lt_r, et_r, ft_r, dt_r,
             wu1_r, bu1_r, wu2_r, bu2_r,
             wit_r, wil_r, wie_r, wif_r, wip_r, wid_r, wav_r, wpg_r,
             bi1_r, wi2_r, bi2_r, out_r):
    f32 = jnp.float32

    def mm(a, b):
        return jax.lax.dot_general(a, b, (((1,), (0,)), ((), ())),
                                   preferred_element_type=f32)

    def small_lookup(idx_col, n, table, wblock):
        cols = lax.broadcasted_iota(jnp.int32, (_BT, n), 1)
        oh = jnp.where(cols == idx_col, 1.0, 0.0).astype(f32)
        return mm(oh, mm(table, wblock))

    hidden_i = (mm(ie_r[...].astype(f32), wit_r[...])
                + mm(pe_r[...].astype(f32), wip_r[...])
                + small_lookup(la_r[...], 64, lt_r[...], wil_r[...])
                + small_lookup(eb_r[...], 8, et_r[...], wie_r[...])
                + small_lookup(fm_r[...], 16, ft_r[...], wif_r[...])
                + small_lookup(de_r[...], 24, dt_r[...], wid_r[...])
                + av_r[...] * wav_r[...]
                + pg_r[...] * wpg_r[...]
                + bi1_r[...])
    hi = hidden_i * jax.nn.sigmoid(hidden_i)
    item_o = mm(hi, wi2_r[...]) + bi2_r[...]

    hu_pre = mm(ue_r[...].astype(f32), wu1_r[...]) + bu1_r[...]
    hu = hu_pre * jax.nn.sigmoid(hu_pre)
    u = mm(hu, wu2_r[...]) + bu2_r[...]

    out_r[...] = jnp.sum(u * item_o, axis=1, keepdims=True)


def _tc_towers(ue, ie, pe, la, eb, fm, de, av, pg,
               lt, et, ft, dt,
               wu1, bu1, wu2, bu2,
               wit, wil, wie, wif, wip, wid, wav, wpg,
               bi1, wi2, bi2):
    bcol = pl.BlockSpec((_BT, 1), lambda i: (i, 0))
    bemb = pl.BlockSpec((_BT, D), lambda i: (i, 0))

    def full(x):
        return pl.BlockSpec(x.shape, lambda i: (0,) * x.ndim)

    in_specs = [bemb, bemb, bemb,
                bcol, bcol, bcol, bcol, bcol, bcol]
    in_specs += [full(x) for x in (lt, et, ft, dt,
                                   wu1, bu1, wu2, bu2,
                                   wit, wil, wie, wif, wip, wid, wav, wpg,
                                   bi1, wi2, bi2)]
    return pl.pallas_call(
        _tc_body,
        grid=(_NB,),
        in_specs=in_specs,
        out_specs=pl.BlockSpec((_BT, 1), lambda i: (i, 0)),
        out_shape=jax.ShapeDtypeStruct((B, 1), jnp.float32),
    )(ue, ie, pe, la, eb, fm, de, av, pg,
      lt, et, ft, dt,
      wu1, bu1, wu2, bu2,
      wit, wil, wie, wif, wip, wid, wav, wpg,
      bi1, wi2, bi2)


def kernel(user_id, item_id, language, is_ebook, format, publisher, pub_decade,
           avg_rating, num_pages,
           user_table, item_table, lang_table, ebook_table, format_table,
           pub_table, decade_table,
           W_u1, b_u1, W_u2, b_u2, W_i1, b_i1, W_i2, b_i2):
    f32 = jnp.float32
    uid2 = user_id.astype(jnp.int32).reshape(_NW, _BPW)
    iid2 = item_id.astype(jnp.int32).reshape(_NW, _BPW)
    pid2 = publisher.astype(jnp.int32).reshape(_NW, _BPW)

    bf16 = jnp.bfloat16
    ue, ie, pe = _sc_gather3(uid2, iid2, pid2,
                             user_table.astype(bf16),
                             item_table.astype(bf16),
                             pub_table.astype(bf16))

    la = language.astype(jnp.int32).reshape(B, 1)
    eb = is_ebook.astype(jnp.int32).reshape(B, 1)
    fm = format.astype(jnp.int32).reshape(B, 1)
    de = pub_decade.astype(jnp.int32).reshape(B, 1)
    av = avg_rating.astype(f32).reshape(B, 1)
    pg = num_pages.astype(f32).reshape(B, 1)

    et = jnp.zeros((8, D), f32).at[:2].set(ebook_table)
    dt = jnp.zeros((24, D), f32).at[:20].set(decade_table)

    wit = W_i1[0:32]
    wil = W_i1[32:64]
    wie = W_i1[64:96]
    wif = W_i1[96:128]
    wip = W_i1[128:160]
    wid = W_i1[160:192]
    wav = W_i1[192:193]
    wpg = W_i1[193:194]

    out = _tc_towers(ue, ie, pe, la, eb, fm, de, av, pg,
                     lang_table, et, format_table, dt,
                     W_u1, b_u1.reshape(1, D), W_u2, b_u2.reshape(1, D),
                     wit, wil, wie, wif, wip, wid, wav, wpg,
                     b_i1.reshape(1, D), W_i2, b_i2.reshape(1, D))
    return out.reshape(B)


# speedup vs baseline: 1.1613x; 1.1613x over previous
import functools

import jax
import jax.numpy as jnp
from jax import lax
from jax.experimental import pallas as pl
from jax.experimental.pallas import tpu as pltpu
from jax.experimental.pallas import tpu_sc as plsc

B = 16384
D = 32

_NC = 2
_NS = 16
_NW = _NC * _NS
_BPW = B // _NW
_CH = 128
_NCH = _BPW // _CH


def _sc_gather3(uid2, iid2, pid2, ut, it, pt):
    dt = ut.dtype
    mesh = plsc.VectorSubcoreMesh(core_axis_name="c", subcore_axis_name="s")
    out_t = jax.ShapeDtypeStruct((B, D), dt)

    @functools.partial(
        pl.kernel,
        out_type=[out_t, out_t, out_t],
        mesh=mesh,
        compiler_params=pltpu.CompilerParams(use_tc_tiling_on_sc=False),
        scratch_types=[
            pltpu.VMEM((_BPW,), jnp.int32),
            pltpu.VMEM((_BPW,), jnp.int32),
            pltpu.VMEM((_BPW,), jnp.int32),
            pltpu.VMEM((_BPW, D), dt),
            pltpu.VMEM((_BPW, D), dt),
            pltpu.VMEM((_BPW, D), dt),
            pltpu.SemaphoreType.DMA,
        ],
    )
    def k(uid_h, iid_h, pid_h, ut_h, it_h, pt_h, ou_h, oi_h, op_h,
          idx_u, idx_i, idx_p, rw_u, rw_i, rw_p, sem):
        wid = lax.axis_index("s") * _NC + lax.axis_index("c")
        base = wid * _BPW
        pltpu.sync_copy(uid_h.at[wid], idx_u)
        pltpu.sync_copy(iid_h.at[wid], idx_i)
        pltpu.sync_copy(pid_h.at[wid], idx_p)
        copies = []
        for j in range(_NCH):
            sl = pl.ds(j * _CH, _CH)
            copies.append(
                pltpu.async_copy(ut_h.at[idx_u.at[sl]], rw_u.at[sl], sem))
            copies.append(
                pltpu.async_copy(it_h.at[idx_i.at[sl]], rw_i.at[sl], sem))
            copies.append(
                pltpu.async_copy(pt_h.at[idx_p.at[sl]], rw_p.at[sl], sem))
        for c in copies:
            c.wait()
        pltpu.sync_copy(rw_u, ou_h.at[pl.ds(base, _BPW)])
        pltpu.sync_copy(rw_i, oi_h.at[pl.ds(base, _BPW)])
        pltpu.sync_copy(rw_p, op_h.at[pl.ds(base, _BPW)])

    return k(uid2, iid2, pid2, ut, it, pt)


_BT = 2048
_NB = B // _BT


def _tc_body(ue_r, ie_r, pe_r,
             la_r, eb_r, fm_r, de_r, av_r, pg_r,
             lt_r, et_r, ft_r, dt_r,
             wu1_r, bu1_r, wu2_r, bu2_r,
             wit_r, wil_r, wie_r, wif_r, wip_r, wid_r, wav_r, wpg_r,
             bi1_r, wi2_r, bi2_r, out_r):
    f32 = jnp.float32

    def mm(a, b):
        return jax.lax.dot_general(a, b, (((1,), (0,)), ((), ())),
                                   preferred_element_type=f32)

    def small_lookup(idx_col, n, table, wblock):
        cols = lax.broadcasted_iota(jnp.int32, (_BT, n), 1)
        oh = jnp.where(cols == idx_col, 1.0, 0.0).astype(f32)
        return mm(oh, mm(table, wblock))

    hidden_i = (mm(ie_r[...].astype(f32), wit_r[...])
                + mm(pe_r[...].astype(f32), wip_r[...])
                + small_lookup(la_r[...], 64, lt_r[...], wil_r[...])
                + small_lookup(eb_r[...], 8, et_r[...], wie_r[...])
                + small_lookup(fm_r[...], 16, ft_r[...], wif_r[...])
                + small_lookup(de_r[...], 24, dt_r[...], wid_r[...])
                + av_r[...] * wav_r[...]
                + pg_r[...] * wpg_r[...]
                + bi1_r[...])
    hi = hidden_i * jax.nn.sigmoid(hidden_i)
    item_o = mm(hi, wi2_r[...]) + bi2_r[...]

    hu_pre = mm(ue_r[...].astype(f32), wu1_r[...]) + bu1_r[...]
    hu = hu_pre * jax.nn.sigmoid(hu_pre)
    u = mm(hu, wu2_r[...]) + bu2_r[...]

    out_r[...] = jnp.sum(u * item_o, axis=1, keepdims=True)


def _tc_towers(ue, ie, pe, la, eb, fm, de, av, pg,
               lt, et, ft, dt,
               wu1, bu1, wu2, bu2,
               wit, wil, wie, wif, wip, wid, wav, wpg,
               bi1, wi2, bi2):
    bcol = pl.BlockSpec((_BT, 1), lambda i: (i, 0))
    bemb = pl.BlockSpec((_BT, D), lambda i: (i, 0))

    def full(x):
        return pl.BlockSpec(x.shape, lambda i: (0,) * x.ndim)

    in_specs = [bemb, bemb, bemb,
                bcol, bcol, bcol, bcol, bcol, bcol]
    in_specs += [full(x) for x in (lt, et, ft, dt,
                                   wu1, bu1, wu2, bu2,
                                   wit, wil, wie, wif, wip, wid, wav, wpg,
                                   bi1, wi2, bi2)]
    return pl.pallas_call(
        _tc_body,
        grid=(_NB,),
        in_specs=in_specs,
        out_specs=pl.BlockSpec((_BT, 1), lambda i: (i, 0)),
        out_shape=jax.ShapeDtypeStruct((B, 1), jnp.float32),
    )(ue, ie, pe, la, eb, fm, de, av, pg,
      lt, et, ft, dt,
      wu1, bu1, wu2, bu2,
      wit, wil, wie, wif, wip, wid, wav, wpg,
      bi1, wi2, bi2)


def kernel(user_id, item_id, language, is_ebook, format, publisher, pub_decade,
           avg_rating, num_pages,
           user_table, item_table, lang_table, ebook_table, format_table,
           pub_table, decade_table,
           W_u1, b_u1, W_u2, b_u2, W_i1, b_i1, W_i2, b_i2):
    f32 = jnp.float32
    uid2 = user_id.astype(jnp.int32).reshape(_NW, _BPW)
    iid2 = item_id.astype(jnp.int32).reshape(_NW, _BPW)
    pid2 = publisher.astype(jnp.int32).reshape(_NW, _BPW)

    ue, ie, pe = _sc_gather3(uid2, iid2, pid2,
                             user_table, item_table, pub_table)

    la = language.astype(jnp.int32).reshape(B, 1)
    eb = is_ebook.astype(jnp.int32).reshape(B, 1)
    fm = format.astype(jnp.int32).reshape(B, 1)
    de = pub_decade.astype(jnp.int32).reshape(B, 1)
    av = avg_rating.astype(f32).reshape(B, 1)
    pg = num_pages.astype(f32).reshape(B, 1)

    et = jnp.zeros((8, D), f32).at[:2].set(ebook_table)
    dt = jnp.zeros((24, D), f32).at[:20].set(decade_table)

    wit = W_i1[0:32]
    wil = W_i1[32:64]
    wie = W_i1[64:96]
    wif = W_i1[96:128]
    wip = W_i1[128:160]
    wid = W_i1[160:192]
    wav = W_i1[192:193]
    wpg = W_i1[193:194]

    out = _tc_towers(ue, ie, pe, la, eb, fm, de, av, pg,
                     lang_table, et, format_table, dt,
                     W_u1, b_u1.reshape(1, D), W_u2, b_u2.reshape(1, D),
                     wit, wil, wie, wif, wip, wid, wav, wpg,
                     b_i1.reshape(1, D), W_i2, b_i2.reshape(1, D))
    return out.reshape(B)


# speedup vs baseline: 1.7939x; 1.5447x over previous
import functools

import jax
import jax.numpy as jnp
from jax import lax
from jax.experimental import pallas as pl
from jax.experimental.pallas import tpu as pltpu
from jax.experimental.pallas import tpu_sc as plsc

B = 16384
D = 32

_NC = 2
_NS = 16
_NW = _NC * _NS
_BPW = B // _NW
_CH = 128
_NCH = _BPW // _CH
_WAVE = 2

_TW = 8192
_TS = _TW // 4


def _tc_detile(tT):
    R = tT.shape[1]
    G = (R + _TW - 1) // _TW

    def body(t_r, i32_r, out_r):
        x = t_r[...]
        pieces = []
        for q in range(4):
            xq = x[:, q * _TS:(q + 1) * _TS]
            pieces.append(jax.lax.dot_general(
                xq, i32_r[...], (((0,), (0,)), ((), ())),
                preferred_element_type=jnp.float32))
        out_r[...] = jnp.concatenate(pieces, axis=1)

    return pl.pallas_call(
        body,
        grid=(G,),
        in_specs=[pl.BlockSpec((D, _TW), lambda i: (0, i)),
                  pl.BlockSpec((D, D), lambda i: (0, 0))],
        out_specs=pl.BlockSpec((_TS, 128), lambda i: (i, 0)),
        out_shape=jax.ShapeDtypeStruct((G * _TS, 128), jnp.float32),
    )(tT, jnp.eye(D, dtype=jnp.float32))


def _sc_gather3(uid2, iid2, pid2, utP, itP, ptP):
    mesh = plsc.VectorSubcoreMesh(core_axis_name="c", subcore_axis_name="s")
    out_t = jax.ShapeDtypeStruct((B, 128), jnp.float32)

    @functools.partial(
        pl.kernel,
        out_type=[out_t, out_t, out_t],
        mesh=mesh,
        scratch_types=[
            pltpu.VMEM((_BPW,), jnp.int32),
            pltpu.VMEM((_BPW,), jnp.int32),
            pltpu.VMEM((_BPW,), jnp.int32),
            pltpu.VMEM((_WAVE * _CH, 128), jnp.float32),
            pltpu.VMEM((_WAVE * _CH, 128), jnp.float32),
            pltpu.VMEM((_WAVE * _CH, 128), jnp.float32),
            pltpu.SemaphoreType.DMA,
        ],
    )
    def k(uid_h, iid_h, pid_h, ut_h, it_h, pt_h, ou_h, oi_h, op_h,
          idx_u, idx_i, idx_p, rw_u, rw_i, rw_p, sem):
        wid = lax.axis_index("s") * _NC + lax.axis_index("c")
        base = wid * _BPW
        pltpu.sync_copy(uid_h.at[wid], idx_u)
        pltpu.sync_copy(iid_h.at[wid], idx_i)
        pltpu.sync_copy(pid_h.at[wid], idx_p)
        for idx in (idx_u, idx_i, idx_p):
            for s in range(_BPW // 16):
                sl = pl.ds(s * 16, 16)
                v = idx[sl]
                idx[sl] = ((v >> 13) << 11) | (v & 2047)
        for w in range(_NCH // _WAVE):
            copies = []
            for b in range(_WAVE):
                j = w * _WAVE + b
                src = pl.ds(j * _CH, _CH)
                dst = pl.ds(b * _CH, _CH)
                copies.append(
                    pltpu.async_copy(ut_h.at[idx_u.at[src]], rw_u.at[dst], sem))
                copies.append(
                    pltpu.async_copy(it_h.at[idx_i.at[src]], rw_i.at[dst], sem))
                copies.append(
                    pltpu.async_copy(pt_h.at[idx_p.at[src]], rw_p.at[dst], sem))
            for c in copies:
                c.wait()
            dst = pl.ds(base + w * _WAVE * _CH, _WAVE * _CH)
            pltpu.sync_copy(rw_u, ou_h.at[dst])
            pltpu.sync_copy(rw_i, oi_h.at[dst])
            pltpu.sync_copy(rw_p, op_h.at[dst])

    return k(uid2, iid2, pid2, utP, itP, ptP)


_BT = 2048
_NB = B // _BT


def _tc_body(ue_r, ie_r, pe_r, su_r, si_r, sp_r,
             la_r, eb_r, fm_r, de_r, av_r, pg_r,
             lt_r, et_r, ft_r, dt_r,
             wu1_r, bu1_r, wu2_r, bu2_r,
             wit_r, wil_r, wie_r, wif_r, wip_r, wid_r, wav_r, wpg_r,
             bi1_r, wi2_r, bi2_r, out_r):
    f32 = jnp.float32

    def mm(a, b):
        return jax.lax.dot_general(a, b, (((1,), (0,)), ((), ())),
                                   preferred_element_type=f32)

    grp = lax.broadcasted_iota(jnp.int32, (_BT, 128), 1) >> 5

    def masked(wide_r, sub_r):
        q = (sub_r[...] >> 11) & 3
        return jnp.where(grp == q, wide_r[...], 0.0)

    def small_lookup(idx_col, n, table, wblock):
        cols = lax.broadcasted_iota(jnp.int32, (_BT, n), 1)
        oh = jnp.where(cols == idx_col, 1.0, 0.0).astype(f32)
        return mm(oh, mm(table, wblock))

    hidden_i = (mm(masked(ie_r, si_r), wit_r[...])
                + mm(masked(pe_r, sp_r), wip_r[...])
                + small_lookup(la_r[...], 64, lt_r[...], wil_r[...])
                + small_lookup(eb_r[...], 8, et_r[...], wie_r[...])
                + small_lookup(fm_r[...], 16, ft_r[...], wif_r[...])
                + small_lookup(de_r[...], 24, dt_r[...], wid_r[...])
                + av_r[...] * wav_r[...]
                + pg_r[...] * wpg_r[...]
                + bi1_r[...])
    hi = hidden_i * jax.nn.sigmoid(hidden_i)
    item_o = mm(hi, wi2_r[...]) + bi2_r[...]

    hu_pre = mm(masked(ue_r, su_r), wu1_r[...]) + bu1_r[...]
    hu = hu_pre * jax.nn.sigmoid(hu_pre)
    u = mm(hu, wu2_r[...]) + bu2_r[...]

    out_r[...] = jnp.sum(u * item_o, axis=1, keepdims=True)


def _tc_towers(ue, ie, pe, su, si, sp, la, eb, fm, de, av, pg,
               lt, et, ft, dt,
               wu1, bu1, wu2, bu2,
               wit, wil, wie, wif, wip, wid, wav, wpg,
               bi1, wi2, bi2):
    bcol = pl.BlockSpec((_BT, 1), lambda i: (i, 0))
    bwide = pl.BlockSpec((_BT, 128), lambda i: (i, 0))

    def full(x):
        return pl.BlockSpec(x.shape, lambda i: (0,) * x.ndim)

    in_specs = [bwide, bwide, bwide, bcol, bcol, bcol,
                bcol, bcol, bcol, bcol, bcol, bcol]
    in_specs += [full(x) for x in (lt, et, ft, dt,
                                   wu1, bu1, wu2, bu2,
                                   wit, wil, wie, wif, wip, wid, wav, wpg,
                                   bi1, wi2, bi2)]
    return pl.pallas_call(
        _tc_body,
        grid=(_NB,),
        in_specs=in_specs,
        out_specs=pl.BlockSpec((_BT, 1), lambda i: (i, 0)),
        out_shape=jax.ShapeDtypeStruct((B, 1), jnp.float32),
    )(ue, ie, pe, su, si, sp, la, eb, fm, de, av, pg,
      lt, et, ft, dt,
      wu1, bu1, wu2, bu2,
      wit, wil, wie, wif, wip, wid, wav, wpg,
      bi1, wi2, bi2)


def kernel(user_id, item_id, language, is_ebook, format, publisher, pub_decade,
           avg_rating, num_pages,
           user_table, item_table, lang_table, ebook_table, format_table,
           pub_table, decade_table,
           W_u1, b_u1, W_u2, b_u2, W_i1, b_i1, W_i2, b_i2):
    f32 = jnp.float32
    uid2 = user_id.astype(jnp.int32).reshape(_NW, _BPW)
    iid2 = item_id.astype(jnp.int32).reshape(_NW, _BPW)
    pid2 = publisher.astype(jnp.int32).reshape(_NW, _BPW)

    utP = _tc_detile(user_table.T)
    itP = _tc_detile(item_table.T)
    ptP = _tc_detile(pub_table.T)

    ue, ie, pe = _sc_gather3(uid2, iid2, pid2, utP, itP, ptP)

    su = user_id.astype(jnp.int32).reshape(B, 1)
    si = item_id.astype(jnp.int32).reshape(B, 1)
    sp = publisher.astype(jnp.int32).reshape(B, 1)
    la = language.astype(jnp.int32).reshape(B, 1)
    eb = is_ebook.astype(jnp.int32).reshape(B, 1)
    fm = format.astype(jnp.int32).reshape(B, 1)
    de = pub_decade.astype(jnp.int32).reshape(B, 1)
    av = avg_rating.astype(f32).reshape(B, 1)
    pg = num_pages.astype(f32).reshape(B, 1)

    et = jnp.zeros((8, D), f32).at[:2].set(ebook_table)
    dt = jnp.zeros((24, D), f32).at[:20].set(decade_table)

    wu1t = jnp.concatenate([W_u1] * 4, axis=0)
    witt = jnp.concatenate([W_i1[0:32]] * 4, axis=0)
    wipt = jnp.concatenate([W_i1[128:160]] * 4, axis=0)

    wil = W_i1[32:64]
    wie = W_i1[64:96]
    wif = W_i1[96:128]
    wid = W_i1[160:192]
    wav = W_i1[192:193]
    wpg = W_i1[193:194]

    out = _tc_towers(ue, ie, pe, su, si, sp, la, eb, fm, de, av, pg,
                     lang_table, et, format_table, dt,
                     wu1t, b_u1.reshape(1, D), W_u2, b_u2.reshape(1, D),
                     witt, wil, wie, wif, wipt, wid, wav, wpg,
                     b_i1.reshape(1, D), W_i2, b_i2.reshape(1, D))
    return out.reshape(B)


# speedup vs baseline: 1.9500x; 1.0870x over previous
import functools

import jax
import jax.numpy as jnp
from jax import lax
from jax.experimental import pallas as pl
from jax.experimental.pallas import tpu as pltpu
from jax.experimental.pallas import tpu_sc as plsc

B = 16384
D = 32

_NC = 2
_NS = 16
_NW = _NC * _NS
_BPW = B // _NW
_CH = 128
_NCH = _BPW // _CH
_WAVE = 2

_TW = 8192
_TS = _TW // 4

_GDT = jnp.float32


def _tc_detile(tT):
    R = tT.shape[1]
    G = (R + _TW - 1) // _TW

    def body(t_r, i32_r, out_r):
        x = t_r[...]
        pieces = []
        for q in range(4):
            xq = x[:, q * _TS:(q + 1) * _TS]
            pieces.append(jax.lax.dot_general(
                xq, i32_r[...], (((0,), (0,)), ((), ())),
                preferred_element_type=jnp.float32))
        out_r[...] = jnp.concatenate(pieces, axis=1).astype(_GDT)

    return pl.pallas_call(
        body,
        grid=(G,),
        in_specs=[pl.BlockSpec((D, _TW), lambda i: (0, i)),
                  pl.BlockSpec((D, D), lambda i: (0, 0))],
        out_specs=pl.BlockSpec((_TS, 128), lambda i: (i, 0)),
        out_shape=jax.ShapeDtypeStruct((G * _TS, 128), _GDT),
    )(tT, jnp.eye(D, dtype=jnp.float32))


def _sc_gather(idx2, tP):
    mesh = plsc.VectorSubcoreMesh(core_axis_name="c", subcore_axis_name="s")

    @functools.partial(
        pl.kernel,
        out_type=jax.ShapeDtypeStruct((B, 128), tP.dtype),
        mesh=mesh,
        scratch_types=[
            pltpu.VMEM((_BPW,), jnp.int32),
            pltpu.VMEM((_WAVE * _CH, 128), tP.dtype),
            pltpu.SemaphoreType.DMA,
        ],
    )
    def k(idx_h, t_h, o_h, idx_v, rw, sem):
        wid = lax.axis_index("s") * _NC + lax.axis_index("c")
        base = wid * _BPW
        pltpu.sync_copy(idx_h.at[wid], idx_v)
        for s in range(_BPW // 16):
            sl = pl.ds(s * 16, 16)
            v = idx_v[sl]
            idx_v[sl] = ((v >> 13) << 11) | (v & 2047)
        for w in range(_NCH // _WAVE):
            copies = []
            for b in range(_WAVE):
                j = w * _WAVE + b
                copies.append(pltpu.async_copy(
                    t_h.at[idx_v.at[pl.ds(j * _CH, _CH)]],
                    rw.at[pl.ds(b * _CH, _CH)], sem))
            for c in copies:
                c.wait()
            pltpu.sync_copy(
                rw, o_h.at[pl.ds(base + w * _WAVE * _CH, _WAVE * _CH)])

    return k(idx2, tP)


_BT = 2048
_NB = B // _BT

_F_SU, _F_SI, _F_SP, _F_LA, _F_EB, _F_FM, _F_DE, _F_AV, _F_PG = range(9)


def _tc_body(ue_r, ie_r, pe_r, fx_r,
             lt_r, et_r, ft_r, dt_r,
             wu1_r, bu1_r, wu2_r, bu2_r,
             wit_r, wil_r, wie_r, wif_r, wip_r, wid_r, wav_r, wpg_r,
             bi1_r, wi2_r, bi2_r, out_r):
    f32 = jnp.float32
    fx = fx_r[...]

    def col(j):
        return fx[:, j:j + 1]

    def icol(j):
        return col(j).astype(jnp.int32)

    def mm(a, b):
        return jax.lax.dot_general(a, b, (((1,), (0,)), ((), ())),
                                   preferred_element_type=f32)

    grp = lax.broadcasted_iota(jnp.int32, (_BT, 128), 1) >> 5

    def masked(wide_r, sub):
        q = (sub >> 11) & 3
        return jnp.where(grp == q, wide_r[...].astype(f32), 0.0)

    def small_lookup(idx_col, n, table, wblock):
        cols = lax.broadcasted_iota(jnp.int32, (_BT, n), 1)
        oh = jnp.where(cols == idx_col, 1.0, 0.0).astype(f32)
        return mm(oh, mm(table, wblock))

    hidden_i = (mm(masked(ie_r, icol(_F_SI)), wit_r[...])
                + mm(masked(pe_r, icol(_F_SP)), wip_r[...])
                + small_lookup(icol(_F_LA), 64, lt_r[...], wil_r[...])
                + small_lookup(icol(_F_EB), 8, et_r[...], wie_r[...])
                + small_lookup(icol(_F_FM), 16, ft_r[...], wif_r[...])
                + small_lookup(icol(_F_DE), 24, dt_r[...], wid_r[...])
                + col(_F_AV) * wav_r[...]
                + col(_F_PG) * wpg_r[...]
                + bi1_r[...])
    hi = hidden_i * jax.nn.sigmoid(hidden_i)
    item_o = mm(hi, wi2_r[...]) + bi2_r[...]

    hu_pre = mm(masked(ue_r, icol(_F_SU)), wu1_r[...]) + bu1_r[...]
    hu = hu_pre * jax.nn.sigmoid(hu_pre)
    u = mm(hu, wu2_r[...]) + bu2_r[...]

    out_r[...] = jnp.sum(u * item_o, axis=1, keepdims=True)


def _tc_towers(ue, ie, pe, fx,
               lt, et, ft, dt,
               wu1, bu1, wu2, bu2,
               wit, wil, wie, wif, wip, wid, wav, wpg,
               bi1, wi2, bi2):
    bwide = pl.BlockSpec((_BT, 128), lambda i: (i, 0))

    def full(x):
        return pl.BlockSpec(x.shape, lambda i: (0,) * x.ndim)

    in_specs = [bwide, bwide, bwide,
                pl.BlockSpec((_BT, 16), lambda i: (i, 0))]
    in_specs += [full(x) for x in (lt, et, ft, dt,
                                   wu1, bu1, wu2, bu2,
                                   wit, wil, wie, wif, wip, wid, wav, wpg,
                                   bi1, wi2, bi2)]
    return pl.pallas_call(
        _tc_body,
        grid=(_NB,),
        in_specs=in_specs,
        out_specs=pl.BlockSpec((_BT, 1), lambda i: (i, 0)),
        out_shape=jax.ShapeDtypeStruct((B, 1), jnp.float32),
    )(ue, ie, pe, fx,
      lt, et, ft, dt,
      wu1, bu1, wu2, bu2,
      wit, wil, wie, wif, wip, wid, wav, wpg,
      bi1, wi2, bi2)


def kernel(user_id, item_id, language, is_ebook, format, publisher, pub_decade,
           avg_rating, num_pages,
           user_table, item_table, lang_table, ebook_table, format_table,
           pub_table, decade_table,
           W_u1, b_u1, W_u2, b_u2, W_i1, b_i1, W_i2, b_i2):
    f32 = jnp.float32
    uid2 = user_id.astype(jnp.int32).reshape(_NW, _BPW)
    iid2 = item_id.astype(jnp.int32).reshape(_NW, _BPW)
    pid2 = publisher.astype(jnp.int32).reshape(_NW, _BPW)

    ue = _sc_gather(uid2, _tc_detile(user_table.T))
    ie = _sc_gather(iid2, _tc_detile(item_table.T))
    pe = _sc_gather(pid2, _tc_detile(pub_table.T))

    fx = jnp.stack(
        [user_id.astype(f32), item_id.astype(f32), publisher.astype(f32),
         language.astype(f32), is_ebook.astype(f32), format.astype(f32),
         pub_decade.astype(f32), avg_rating.astype(f32),
         num_pages.astype(f32)] + [jnp.zeros((B,), f32)] * 7,
        axis=1)

    et = jnp.zeros((8, D), f32).at[:2].set(ebook_table)
    dt = jnp.zeros((24, D), f32).at[:20].set(decade_table)

    wu1t = jnp.concatenate([W_u1] * 4, axis=0)
    witt = jnp.concatenate([W_i1[0:32]] * 4, axis=0)
    wipt = jnp.concatenate([W_i1[128:160]] * 4, axis=0)

    wil = W_i1[32:64]
    wie = W_i1[64:96]
    wif = W_i1[96:128]
    wid = W_i1[160:192]
    wav = W_i1[192:193]
    wpg = W_i1[193:194]

    out = _tc_towers(ue, ie, pe, fx,
                     lang_table, et, format_table, dt,
                     wu1t, b_u1.reshape(1, D), W_u2, b_u2.reshape(1, D),
                     witt, wil, wie, wif, wipt, wid, wav, wpg,
                     b_i1.reshape(1, D), W_i2, b_i2.reshape(1, D))
    return out.reshape(B)


# speedup vs baseline: 1.9777x; 1.0142x over previous
import functools

import jax
import jax.numpy as jnp
from jax import lax
from jax.experimental import pallas as pl
from jax.experimental.pallas import tpu as pltpu
from jax.experimental.pallas import tpu_sc as plsc

B = 16384
D = 32

_NC = 2
_NS = 16
_NW = _NC * _NS
_BPW = B // _NW
_CH = 128
_NCH = _BPW // _CH
_WAVE = 2

_TW = 16384
_TS = _TW // 4
_SH = 14
_SQ = 12
_MS = _TS - 1

_GDT = jnp.float32


def _tc_detile(tT):
    R = tT.shape[1]
    G = (R + _TW - 1) // _TW

    def body(t_r, i32_r, out_r):
        x = t_r[...]
        pieces = []
        for q in range(4):
            xq = x[:, q * _TS:(q + 1) * _TS]
            pieces.append(jax.lax.dot_general(
                xq, i32_r[...], (((0,), (0,)), ((), ())),
                preferred_element_type=jnp.float32))
        out_r[...] = jnp.concatenate(pieces, axis=1).astype(_GDT)

    return pl.pallas_call(
        body,
        grid=(G,),
        in_specs=[pl.BlockSpec((D, _TW), lambda i: (0, i)),
                  pl.BlockSpec((D, D), lambda i: (0, 0))],
        out_specs=pl.BlockSpec((_TS, 128), lambda i: (i, 0)),
        out_shape=jax.ShapeDtypeStruct((G * _TS, 128), _GDT),
    )(tT, jnp.eye(D, dtype=jnp.float32))


def _sc_gather(idx2, tP):
    mesh = plsc.VectorSubcoreMesh(core_axis_name="c", subcore_axis_name="s")

    @functools.partial(
        pl.kernel,
        out_type=jax.ShapeDtypeStruct((B, 128), tP.dtype),
        mesh=mesh,
        scratch_types=[
            pltpu.VMEM((_BPW,), jnp.int32),
            pltpu.VMEM((_WAVE * _CH, 128), tP.dtype),
            pltpu.SemaphoreType.DMA,
        ],
    )
    def k(idx_h, t_h, o_h, idx_v, rw, sem):
        wid = lax.axis_index("s") * _NC + lax.axis_index("c")
        base = wid * _BPW
        pltpu.sync_copy(idx_h.at[wid], idx_v)
        for s in range(_BPW // 16):
            sl = pl.ds(s * 16, 16)
            v = idx_v[sl]
            idx_v[sl] = ((v >> _SH) << _SQ) | (v & _MS)
        for w in range(_NCH // _WAVE):
            copies = []
            for b in range(_WAVE):
                j = w * _WAVE + b
                copies.append(pltpu.async_copy(
                    t_h.at[idx_v.at[pl.ds(j * _CH, _CH)]],
                    rw.at[pl.ds(b * _CH, _CH)], sem))
            for c in copies:
                c.wait()
            pltpu.sync_copy(
                rw, o_h.at[pl.ds(base + w * _WAVE * _CH, _WAVE * _CH)])

    return k(idx2, tP)


_BT = 2048
_NB = B // _BT

_F_SU, _F_SI, _F_SP, _F_LA, _F_EB, _F_FM, _F_DE, _F_AV, _F_PG = range(9)


def _tc_body(ue_r, ie_r, pe_r, fx_r,
             lt_r, et_r, ft_r, dt_r,
             wu1_r, bu1_r, wu2_r, bu2_r,
             wit_r, wil_r, wie_r, wif_r, wip_r, wid_r, wav_r, wpg_r,
             bi1_r, wi2_r, bi2_r, out_r):
    f32 = jnp.float32
    fx = fx_r[...]

    def col(j):
        return fx[:, j:j + 1]

    def icol(j):
        return col(j).astype(jnp.int32)

    def mm(a, b):
        return jax.lax.dot_general(a, b, (((1,), (0,)), ((), ())),
                                   preferred_element_type=f32)

    grp = lax.broadcasted_iota(jnp.int32, (_BT, 128), 1) >> 5

    def masked(wide_r, sub):
        q = (sub >> _SQ) & 3
        return jnp.where(grp == q, wide_r[...].astype(f32), 0.0)

    def small_lookup(idx_col, n, table, wblock):
        cols = lax.broadcasted_iota(jnp.int32, (_BT, n), 1)
        oh = jnp.where(cols == idx_col, 1.0, 0.0).astype(f32)
        return mm(oh, mm(table, wblock))

    hidden_i = (mm(masked(ie_r, icol(_F_SI)), wit_r[...])
                + mm(masked(pe_r, icol(_F_SP)), wip_r[...])
                + small_lookup(icol(_F_LA), 64, lt_r[...], wil_r[...])
                + small_lookup(icol(_F_EB), 8, et_r[...], wie_r[...])
                + small_lookup(icol(_F_FM), 16, ft_r[...], wif_r[...])
                + small_lookup(icol(_F_DE), 24, dt_r[...], wid_r[...])
                + col(_F_AV) * wav_r[...]
                + col(_F_PG) * wpg_r[...]
                + bi1_r[...])
    hi = hidden_i * jax.nn.sigmoid(hidden_i)
    item_o = mm(hi, wi2_r[...]) + bi2_r[...]

    hu_pre = mm(masked(ue_r, icol(_F_SU)), wu1_r[...]) + bu1_r[...]
    hu = hu_pre * jax.nn.sigmoid(hu_pre)
    u = mm(hu, wu2_r[...]) + bu2_r[...]

    out_r[...] = jnp.sum(u * item_o, axis=1, keepdims=True)


def _tc_towers(ue, ie, pe, fx,
               lt, et, ft, dt,
               wu1, bu1, wu2, bu2,
               wit, wil, wie, wif, wip, wid, wav, wpg,
               bi1, wi2, bi2):
    bwide = pl.BlockSpec((_BT, 128), lambda i: (i, 0))

    def full(x):
        return pl.BlockSpec(x.shape, lambda i: (0,) * x.ndim)

    in_specs = [bwide, bwide, bwide,
                pl.BlockSpec((_BT, 16), lambda i: (i, 0))]
    in_specs += [full(x) for x in (lt, et, ft, dt,
                                   wu1, bu1, wu2, bu2,
                                   wit, wil, wie, wif, wip, wid, wav, wpg,
                                   bi1, wi2, bi2)]
    return pl.pallas_call(
        _tc_body,
        grid=(_NB,),
        in_specs=in_specs,
        out_specs=pl.BlockSpec((_BT, 1), lambda i: (i, 0)),
        out_shape=jax.ShapeDtypeStruct((B, 1), jnp.float32),
    )(ue, ie, pe, fx,
      lt, et, ft, dt,
      wu1, bu1, wu2, bu2,
      wit, wil, wie, wif, wip, wid, wav, wpg,
      bi1, wi2, bi2)


def kernel(user_id, item_id, language, is_ebook, format, publisher, pub_decade,
           avg_rating, num_pages,
           user_table, item_table, lang_table, ebook_table, format_table,
           pub_table, decade_table,
           W_u1, b_u1, W_u2, b_u2, W_i1, b_i1, W_i2, b_i2):
    f32 = jnp.float32
    uid2 = user_id.astype(jnp.int32).reshape(_NW, _BPW)
    iid2 = item_id.astype(jnp.int32).reshape(_NW, _BPW)
    pid2 = publisher.astype(jnp.int32).reshape(_NW, _BPW)

    ue = _sc_gather(uid2, _tc_detile(user_table.T))
    ie = _sc_gather(iid2, _tc_detile(item_table.T))
    pe = _sc_gather(pid2, _tc_detile(pub_table.T))

    fx = jnp.stack(
        [user_id.astype(f32), item_id.astype(f32), publisher.astype(f32),
         language.astype(f32), is_ebook.astype(f32), format.astype(f32),
         pub_decade.astype(f32), avg_rating.astype(f32),
         num_pages.astype(f32)] + [jnp.zeros((B,), f32)] * 7,
        axis=1)

    et = jnp.zeros((8, D), f32).at[:2].set(ebook_table)
    dt = jnp.zeros((24, D), f32).at[:20].set(decade_table)

    wu1t = jnp.concatenate([W_u1] * 4, axis=0)
    witt = jnp.concatenate([W_i1[0:32]] * 4, axis=0)
    wipt = jnp.concatenate([W_i1[128:160]] * 4, axis=0)

    wil = W_i1[32:64]
    wie = W_i1[64:96]
    wif = W_i1[96:128]
    wid = W_i1[160:192]
    wav = W_i1[192:193]
    wpg = W_i1[193:194]

    out = _tc_towers(ue, ie, pe, fx,
                     lang_table, et, format_table, dt,
                     wu1t, b_u1.reshape(1, D), W_u2, b_u2.reshape(1, D),
                     witt, wil, wie, wif, wipt, wid, wav, wpg,
                     b_i1.reshape(1, D), W_i2, b_i2.reshape(1, D))
    return out.reshape(B)
